# baseline (speedup 1.0000x reference)
import functools

import jax
import jax.numpy as jnp
from jax import lax
from jax.experimental import pallas as pl
from jax.experimental.pallas import tpu as pltpu
from jax.experimental.pallas import tpu_sc as plsc

L = 16
NC = 2
NS = 16
NW = NC * NS
BATCH = 16384
BPW = BATCH // NW
CHUNK = 32
NCHUNK = BPW // CHUNK
GPC = CHUNK // L
EMB = 64
RG = 8

_MESH = plsc.VectorSubcoreMesh(
    core_axis_name="c", subcore_axis_name="s", num_cores=NC, num_subcores=NS
)


def _inv_norm(n2):
    i = plsc.bitcast(n2, jnp.int32)
    y = plsc.bitcast(0x5F3759DF - (i >> 1), jnp.float32)
    for _ in range(3):
        y = y * (1.5 - 0.5 * n2 * y * y)
    norm = n2 * y
    return jnp.where(norm > 1e-12, y, jnp.float32(1e12))


@functools.partial(
    pl.kernel,
    out_type=jax.ShapeDtypeStruct((BATCH,), jnp.float32),
    mesh=_MESH,
    compiler_params=pltpu.CompilerParams(
        needs_layout_passes=False, use_tc_tiling_on_sc=True
    ),
    scratch_types=[
        pltpu.VMEM((BPW,), jnp.int32),
        pltpu.VMEM((BPW,), jnp.int32),
        pltpu.VMEM((BPW,), jnp.int32),
        pltpu.VMEM((CHUNK, RG, EMB), jnp.float32),
        pltpu.VMEM((CHUNK, RG, EMB), jnp.float32),
        pltpu.VMEM((CHUNK, RG, EMB), jnp.float32),
        pltpu.VMEM((BPW,), jnp.float32),
        pltpu.SemaphoreType.DMA,
    ],
)
def _sc_kernel(hidx_hbm, ridx_hbm, tidx_hbm, ent_hbm, rel_hbm, out_hbm,
               hi_v, ri_v, ti_v, hbuf, rbuf, tbuf, res_v, sem):
    wid = lax.axis_index("s") * NC + lax.axis_index("c")
    base = wid * BPW
    pltpu.sync_copy(hidx_hbm.at[pl.ds(base, BPW)], hi_v)
    pltpu.sync_copy(ridx_hbm.at[pl.ds(base, BPW)], ri_v)
    pltpu.sync_copy(tidx_hbm.at[pl.ds(base, BPW)], ti_v)

    zeros = jnp.zeros((L,), jnp.float32)

    def chunk_body(c, carry):
        for vg in range(GPC):
            sl = pl.ds(c * CHUNK + vg * L, L)
            hv16 = hi_v[sl] >> 3
            rv16 = ri_v[sl] >> 3
            tv16 = ti_v[sl] >> 3
            for i in range(L):
                e = vg * L + i
                pltpu.async_copy(ent_hbm.at[hv16[i]], hbuf.at[e], sem)
                pltpu.async_copy(
                    rel_hbm.at[pl.ds(pl.multiple_of(rv16[i] * RG, RG), RG)],
                    rbuf.at[e], sem)
                pltpu.async_copy(ent_hbm.at[tv16[i]], tbuf.at[e], sem)
        for e in range(CHUNK):
            pltpu.make_async_copy(ent_hbm.at[0], hbuf.at[e], sem).wait()
            pltpu.make_async_copy(rel_hbm.at[pl.ds(0, RG)], rbuf.at[e], sem).wait()
            pltpu.make_async_copy(ent_hbm.at[0], tbuf.at[e], sem).wait()

        def group(g, carry2):
            s = c * CHUNK + g * L
            rid = lax.iota(jnp.int32, L) + g * L
            sub_h = hi_v[pl.ds(s, L)] & (RG - 1)
            sub_r = ri_v[pl.ds(s, L)] & (RG - 1)
            sub_t = ti_v[pl.ds(s, L)] & (RG - 1)
            acc_h = zeros
            acc_r = zeros
            acc_t = zeros
            for d in range(EMB):
                dv = jnp.full((L,), d, jnp.int32)
                hv = plsc.load_gather(hbuf, [rid, sub_h, dv])
                rv = plsc.load_gather(rbuf, [rid, sub_r, dv])
                tv = plsc.load_gather(tbuf, [rid, sub_t, dv])
                acc_h = acc_h + hv * hv
                acc_r = acc_r + rv * rv
                acc_t = acc_t + tv * tv
            ih = _inv_norm(acc_h)
            ir = _inv_norm(acc_r)
            it = _inv_norm(acc_t)
            acc = zeros
            for d in range(EMB):
                dv = jnp.full((L,), d, jnp.int32)
                hv = plsc.load_gather(hbuf, [rid, sub_h, dv])
                rv = plsc.load_gather(rbuf, [rid, sub_r, dv])
                tv = plsc.load_gather(tbuf, [rid, sub_t, dv])
                acc = acc + jnp.abs(hv * ih + rv * ir - tv * it)
            res_v[pl.ds(s, L)] = acc
            return carry2

        lax.fori_loop(0, GPC, group, 0)
        return carry

    lax.fori_loop(0, NCHUNK, chunk_body, 0)
    pltpu.sync_copy(res_v, out_hbm.at[pl.ds(base, BPW)])


def kernel(edge, entity_embedding, relation_embedding):
    edge = edge.astype(jnp.int32)
    ent3 = entity_embedding.reshape(125000, RG, EMB)
    return _sc_kernel(edge[:, 0], edge[:, 1], edge[:, 2], ent3,
                      relation_embedding)


# speedup vs baseline: 1.1632x; 1.1632x over previous
import functools

import jax
import jax.numpy as jnp
from jax import lax
from jax.experimental import pallas as pl
from jax.experimental.pallas import tpu as pltpu
from jax.experimental.pallas import tpu_sc as plsc

L = 16
NC = 2
NS = 16
NW = NC * NS
BATCH = 16384
BPW = BATCH // NW
CHUNK = 16
NCHUNK = BPW // CHUNK
EMB = 64
RG = 8
NGRP = 125000

_MESH = plsc.VectorSubcoreMesh(
    core_axis_name="c", subcore_axis_name="s", num_cores=NC, num_subcores=NS
)


def _inv_norm(n2):
    i = plsc.bitcast(n2, jnp.int32)
    y = plsc.bitcast(0x5F3759DF - (i >> 1), jnp.float32)
    for _ in range(3):
        y = y * (1.5 - 0.5 * n2 * y * y)
    norm = n2 * y
    return jnp.where(norm > 1e-12, y, jnp.float32(1e12))


@functools.partial(
    pl.kernel,
    out_type=jax.ShapeDtypeStruct((BATCH,), jnp.float32),
    mesh=_MESH,
    compiler_params=pltpu.CompilerParams(
        needs_layout_passes=False, use_tc_tiling_on_sc=True
    ),
    scratch_types=[
        pltpu.VMEM((BPW,), jnp.int32),
        pltpu.VMEM((BPW,), jnp.int32),
        pltpu.VMEM((BPW,), jnp.int32),
        pltpu.VMEM((2, CHUNK, RG, EMB), jnp.float32),
        pltpu.VMEM((2, CHUNK, RG, EMB), jnp.float32),
        pltpu.VMEM((2, CHUNK, RG, EMB), jnp.float32),
        pltpu.VMEM((BPW,), jnp.float32),
        pltpu.SemaphoreType.DMA,
        pltpu.SemaphoreType.DMA,
    ],
)
def _sc_kernel(hidx_hbm, ridx_hbm, tidx_hbm, ent_hbm, rel_hbm, out_hbm,
               hi_v, ri_v, ti_v, hbuf, rbuf, tbuf, res_v, semA, semB):
    wid = lax.axis_index("s") * NC + lax.axis_index("c")
    base = wid * BPW
    pltpu.sync_copy(hidx_hbm.at[pl.ds(base, BPW)], hi_v)
    pltpu.sync_copy(ridx_hbm.at[pl.ds(base, BPW)], ri_v)
    pltpu.sync_copy(tidx_hbm.at[pl.ds(base, BPW)], ti_v)

    zeros = jnp.zeros((L,), jnp.float32)

    def issue(cidx, b, sem):
        sl = pl.ds(cidx * CHUNK, L)
        hv16 = hi_v[sl] >> 3
        rv16 = ri_v[sl] >> 3
        tv16 = ti_v[sl] >> 3
        for i in range(L):
            pltpu.async_copy(ent_hbm.at[hv16[i]], hbuf.at[b, i], sem)
            pltpu.async_copy(rel_hbm.at[rv16[i]], rbuf.at[b, i], sem)
            pltpu.async_copy(ent_hbm.at[tv16[i]], tbuf.at[b, i], sem)

    def drain(b, sem):
        pltpu.make_async_copy(ent_hbm.at[pl.ds(0, CHUNK)], hbuf.at[b], sem).wait()
        pltpu.make_async_copy(rel_hbm.at[pl.ds(0, CHUNK)], rbuf.at[b], sem).wait()
        pltpu.make_async_copy(ent_hbm.at[pl.ds(0, CHUNK)], tbuf.at[b], sem).wait()

    def compute(cidx, b):
        s = cidx * CHUNK
        rid = lax.iota(jnp.int32, L)
        bv = jnp.full((L,), b, jnp.int32)
        sub_h = hi_v[pl.ds(s, L)] & (RG - 1)
        sub_r = ri_v[pl.ds(s, L)] & (RG - 1)
        sub_t = ti_v[pl.ds(s, L)] & (RG - 1)
        acc_h = zeros
        acc_r = zeros
        acc_t = zeros
        for d in range(EMB):
            dv = jnp.full((L,), d, jnp.int32)
            hv = plsc.load_gather(hbuf, [bv, rid, sub_h, dv])
            rv = plsc.load_gather(rbuf, [bv, rid, sub_r, dv])
            tv = plsc.load_gather(tbuf, [bv, rid, sub_t, dv])
            acc_h = acc_h + hv * hv
            acc_r = acc_r + rv * rv
            acc_t = acc_t + tv * tv
        ih = _inv_norm(acc_h)
        ir = _inv_norm(acc_r)
        it = _inv_norm(acc_t)
        acc = zeros
        for d in range(EMB):
            dv = jnp.full((L,), d, jnp.int32)
            hv = plsc.load_gather(hbuf, [bv, rid, sub_h, dv])
            rv = plsc.load_gather(rbuf, [bv, rid, sub_r, dv])
            tv = plsc.load_gather(tbuf, [bv, rid, sub_t, dv])
            acc = acc + jnp.abs(hv * ih + rv * ir - tv * it)
        res_v[pl.ds(s, L)] = acc

    issue(0, 0, semA)

    def pair_body(cc, carry):
        c0 = cc * 2
        issue(c0 + 1, 1, semB)
        drain(0, semA)
        compute(c0, 0)
        issue(jnp.minimum(c0 + 2, NCHUNK - 1), 0, semA)
        drain(1, semB)
        compute(c0 + 1, 1)
        return carry

    lax.fori_loop(0, NCHUNK // 2, pair_body, 0)
    drain(0, semA)

    pltpu.sync_copy(res_v, out_hbm.at[pl.ds(base, BPW)])


def kernel(edge, entity_embedding, relation_embedding):
    edge = edge.astype(jnp.int32)
    ent3 = entity_embedding.reshape(NGRP, RG, EMB)
    rel3 = relation_embedding.reshape(NGRP, RG, EMB)
    return _sc_kernel(edge[:, 0], edge[:, 1], edge[:, 2], ent3, rel3)
